# Initial kernel scaffold; baseline (speedup 1.0000x reference)
import jax
import jax.numpy as jnp
from jax.experimental import pallas as pl


def kernel(x, Wqk, Wv, Wout):
    raise NotImplementedError("write your pallas kernel here")


# speedup vs baseline: 3.2748x; 3.2748x over previous
import jax
import jax.numpy as jnp
from jax import lax
from jax.experimental import pallas as pl
from jax.experimental.pallas import tpu as pltpu
from jax.experimental.pallas import tpu_sc as plsc

B, S, D = 1, 2048, 1024
H, DH = 16, 64
CHUNK_LEN = 128
N_HASHES = 2
N_BUCKETS = 32
NCH = (N_HASHES * S) // CHUNK_LEN
NC, NS, L = 2, 16, 16
PK = 2 * DH

_F32 = jnp.float32
_I32 = jnp.int32
_SC_PARAMS = pltpu.CompilerParams(needs_layout_passes=False)


def _proj_hash_body(x_ref, wqk_ref, wv_ref, rot_ref, qkv_ref, bkt_ref):
    x = x_ref[...]
    qk = jnp.dot(x, wqk_ref[0], preferred_element_type=_F32)
    v = jnp.dot(x, wv_ref[0], preferred_element_type=_F32)
    qkv_ref[0, :, 0:DH] = qk
    qkv_ref[0, :, DH:PK] = v
    rr = jnp.dot(qk, rot_ref[0], preferred_element_type=_F32)
    iota = lax.broadcasted_iota(_I32, (S, N_BUCKETS), 1)
    for hh in range(N_HASHES):
        r = rr[:, hh * 16:(hh + 1) * 16]
        full = jnp.concatenate([r, -r], axis=1)
        m = jnp.max(full, axis=1, keepdims=True)
        idx = jnp.min(jnp.where(full == m, iota, N_BUCKETS), axis=1,
                      keepdims=True)
        bkt_ref[0, :, hh:hh + 1] = idx


def _proj_hash(x2, Wqk, Wv, rot):
    return pl.pallas_call(
        _proj_hash_body,
        grid=(H,),
        in_specs=[
            pl.BlockSpec((S, D), lambda h: (0, 0)),
            pl.BlockSpec((1, D, DH), lambda h: (h, 0, 0)),
            pl.BlockSpec((1, D, DH), lambda h: (h, 0, 0)),
            pl.BlockSpec((1, DH, N_HASHES * 16), lambda h: (h, 0, 0)),
        ],
        out_specs=[
            pl.BlockSpec((1, S, PK), lambda h: (h, 0, 0)),
            pl.BlockSpec((1, S, N_HASHES), lambda h: (h, 0, 0)),
        ],
        out_shape=[
            jax.ShapeDtypeStruct((H, S, PK), _F32),
            jax.ShapeDtypeStruct((H, S, N_HASHES), _I32),
        ],
    )(x2, Wqk, Wv, rot)


def _sort_gather_body(bkt_hbm, qkv_hbm,
                      sqkv_hbm, st_hbm, pos_hbm,
                      b_v, st_v, pos_v, idx_v, buf, sem):
    c = lax.axis_index("c")
    s = lax.axis_index("s")
    w = c * NS + s
    h = w // N_HASHES
    hh = w % N_HASHES
    pltpu.sync_copy(bkt_hbm.at[h, hh], b_v)
    iota16 = lax.broadcasted_iota(_I32, (L,), 0)

    def sort_body(i, ptr):
        bucket = i >> 7
        chunk = i & 127
        bv = b_v[pl.ds(chunk * L, L)]
        msk = bv == bucket
        cnts = jnp.where(msk, 1, 0)
        incl = plsc.cumsum(cnts)
        tgt = jnp.where(msk, ptr + incl - 1, S + iota16)
        plsc.store_scatter(st_v, [tgt], chunk * L + iota16)
        return ptr + jnp.sum(cnts)

    lax.fori_loop(0, N_BUCKETS * (S // L), sort_body, jnp.int32(0))

    def pos_body(ci, carry):
        svec = st_v[pl.ds(ci * L, L)]
        plsc.store_scatter(pos_v, [svec], ci * L + iota16)
        idx_v[pl.ds(ci * L, L)] = svec + h * S
        return carry

    lax.fori_loop(0, S // L, pos_body, 0)

    pltpu.sync_copy(st_v.at[pl.ds(0, S)], st_hbm.at[h, hh])
    pltpu.sync_copy(pos_v, pos_hbm.at[h, hh])

    def gath_body(g, carry):
        idxs = idx_v.at[pl.ds(g * 128, 128)]
        pltpu.async_copy(qkv_hbm.at[idxs], buf, sem).wait()
        pltpu.sync_copy(buf, sqkv_hbm.at[h, pl.ds(hh * S + g * 128, 128)])
        return carry

    lax.fori_loop(0, S // 128, gath_body, 0)


def _sort_gather(bkt, qkvflat):
    mesh = plsc.VectorSubcoreMesh(core_axis_name="c", subcore_axis_name="s",
                                  num_cores=NC, num_subcores=NS)
    fn = pl.kernel(
        _sort_gather_body,
        out_type=[
            jax.ShapeDtypeStruct((H, N_HASHES * S, PK), _F32),
            jax.ShapeDtypeStruct((H, N_HASHES, S), _I32),
            jax.ShapeDtypeStruct((H, N_HASHES, S), _I32),
        ],
        mesh=mesh,
        scratch_types=[
            pltpu.VMEM((S,), _I32),
            pltpu.VMEM((S + L,), _I32),
            pltpu.VMEM((S,), _I32),
            pltpu.VMEM((S,), _I32),
            pltpu.VMEM((128, PK), _F32),
            pltpu.SemaphoreType.DMA,
        ],
        compiler_params=_SC_PARAMS,
    )
    return fn(bkt, qkvflat)


def _attn_body(qc_ref, qp_ref, kc_ref, kp_ref, qi_ref, sol_ref):
    cur = qc_ref[0]
    prv = qp_ref[0]
    q = cur[:, 0:DH]
    qprev = prv[:, 0:DH]
    inv_sqrt_dh = 1.0 / (DH ** 0.5)

    def norm(t):
        return t * lax.rsqrt(jnp.mean(t * t, axis=1, keepdims=True) + 1e-6) \
            * inv_sqrt_dh

    kv = jnp.concatenate([norm(qprev), norm(q)], axis=0)
    bv = jnp.concatenate([prv[:, DH:PK], cur[:, DH:PK]], axis=0)
    dots = lax.dot_general(q, kv, (((1,), (1,)), ((), ())),
                           preferred_element_type=_F32)
    qi = qi_ref[0, 0]
    ki = jnp.concatenate([kp_ref[0, 0], kc_ref[0, 0]], axis=1)
    dots = dots - 1e9 * (qi < ki).astype(_F32) - 1e5 * (qi == ki).astype(_F32)
    m = jnp.max(dots, axis=1, keepdims=True)
    lse = m + jnp.log(jnp.sum(jnp.exp(dots - m), axis=1, keepdims=True))
    p = jnp.exp(dots - lse)
    sol_ref[0, 0, :, 0:DH] = jnp.dot(p, bv, preferred_element_type=_F32)
    sol_ref[0, 0, :, DH:PK] = jnp.broadcast_to(lse, (CHUNK_LEN, DH))


def _attention(sqkv, st_row, st_col):
    cur = lambda h, c: (h, c, 0)
    prev = lambda h, c: (h, (c + NCH - 1) % NCH, 0)
    prev4 = lambda h, c: (h, (c + NCH - 1) % NCH, 0, 0)
    return pl.pallas_call(
        _attn_body,
        grid=(H, NCH),
        in_specs=[
            pl.BlockSpec((1, CHUNK_LEN, PK), cur),
            pl.BlockSpec((1, CHUNK_LEN, PK), prev),
            pl.BlockSpec((1, 1, 1, CHUNK_LEN), lambda h, c: (h, c, 0, 0)),
            pl.BlockSpec((1, 1, 1, CHUNK_LEN), prev4),
            pl.BlockSpec((1, 1, CHUNK_LEN, 1), lambda h, c: (h, c, 0, 0)),
        ],
        out_specs=[
            pl.BlockSpec((1, 1, CHUNK_LEN, PK), lambda h, c: (h, c, 0, 0)),
        ],
        out_shape=[
            jax.ShapeDtypeStruct((H, NCH, CHUNK_LEN, PK), _F32),
        ],
    )(sqkv, sqkv, st_row, st_row, st_col)[0]


def _unsort_body(sol_hbm, pos_hbm, o_hbm,
                 pos_v, idx_v, buf, sem):
    c = lax.axis_index("c")
    s = lax.axis_index("s")
    w = c * NS + s
    h = w // N_HASHES
    hh = w % N_HASHES
    pltpu.sync_copy(pos_hbm.at[h, hh], pos_v)
    base = h * (N_HASHES * S) + hh * S

    def idx_body(ci, carry):
        idx_v[pl.ds(ci * L, L)] = pos_v[pl.ds(ci * L, L)] + base
        return carry

    lax.fori_loop(0, S // L, idx_body, 0)

    def gath_body(g, carry):
        idxs = idx_v.at[pl.ds(g * 128, 128)]
        pltpu.async_copy(sol_hbm.at[idxs], buf, sem).wait()
        pltpu.sync_copy(buf, o_hbm.at[h, hh, pl.ds(g * 128, 128)])
        return carry

    lax.fori_loop(0, S // 128, gath_body, 0)


def _unsort(solflat, pos):
    mesh = plsc.VectorSubcoreMesh(core_axis_name="c", subcore_axis_name="s",
                                  num_cores=NC, num_subcores=NS)
    fn = pl.kernel(
        _unsort_body,
        out_type=[
            jax.ShapeDtypeStruct((H, N_HASHES, S, PK), _F32),
        ],
        mesh=mesh,
        scratch_types=[
            pltpu.VMEM((S,), _I32),
            pltpu.VMEM((S,), _I32),
            pltpu.VMEM((128, PK), _F32),
            pltpu.SemaphoreType.DMA,
        ],
        compiler_params=_SC_PARAMS,
    )
    return fn(solflat, pos)[0]


def _combine_body(o_ref, wout_ref, out_ref):
    h = pl.program_id(0)
    l0 = o_ref[0, 0, :, DH:DH + 1]
    l1 = o_ref[0, 1, :, DH:DH + 1]
    m = jnp.maximum(l0, l1)
    lse = m + jnp.log(jnp.exp(l0 - m) + jnp.exp(l1 - m))
    p0 = jnp.exp(l0 - lse)
    p1 = jnp.exp(l1 - lse)
    wsum = o_ref[0, 0, :, 0:DH] * p0 + o_ref[0, 1, :, 0:DH] * p1
    contrib = jnp.dot(wsum, wout_ref[0], preferred_element_type=_F32)

    @pl.when(h == 0)
    def _():
        out_ref[...] = jnp.zeros_like(out_ref)

    out_ref[...] += contrib


def _combine(o, Wout):
    return pl.pallas_call(
        _combine_body,
        grid=(H,),
        in_specs=[
            pl.BlockSpec((1, N_HASHES, S, PK), lambda h: (h, 0, 0, 0)),
            pl.BlockSpec((1, DH, D), lambda h: (h, 0, 0)),
        ],
        out_specs=pl.BlockSpec((S, D), lambda h: (0, 0)),
        out_shape=jax.ShapeDtypeStruct((S, D), _F32),
    )(o, Wout)


def kernel(x, Wqk, Wv, Wout):
    rot = jax.random.normal(jax.random.key(1),
                            (H, DH, N_HASHES, N_BUCKETS // 2),
                            dtype=_F32).reshape(H, DH, N_HASHES * 16)
    x2 = x[0]
    qkv, bkt = _proj_hash(x2, Wqk, Wv, rot)
    bkt_t = jnp.transpose(bkt, (0, 2, 1))
    sqkv, st, pos = _sort_gather(bkt_t, qkv.reshape(H * S, PK))
    stf = st.astype(_F32)
    st_row = stf.reshape(H, NCH, 1, CHUNK_LEN)
    st_col = stf.reshape(H, NCH, CHUNK_LEN, 1)
    sol = _attention(sqkv, st_row, st_col)
    o = _unsort(sol.reshape(H * N_HASHES * S, PK), pos)
    out = _combine(o, Wout)
    return out.reshape(B, S, D)


# speedup vs baseline: 6.0612x; 1.8509x over previous
import jax
import jax.numpy as jnp
from jax import lax
from jax.experimental import pallas as pl
from jax.experimental.pallas import tpu as pltpu
from jax.experimental.pallas import tpu_sc as plsc

B, S, D = 1, 2048, 1024
H, DH = 16, 64
CHUNK_LEN = 128
N_HASHES = 2
N_BUCKETS = 32
NCH = (N_HASHES * S) // CHUNK_LEN
NC, NS, L = 2, 16, 16
PK = 2 * DH

_F32 = jnp.float32
_I32 = jnp.int32
_SC_PARAMS = pltpu.CompilerParams(needs_layout_passes=False)


def _proj_body(x_ref, w_ref, out_ref):
    out_ref[...] = jnp.dot(x_ref[...], w_ref[...],
                           preferred_element_type=_F32)


def _proj(x2, Wcat):
    ntile = 4
    tile = H * PK // ntile
    return pl.pallas_call(
        _proj_body,
        grid=(ntile,),
        in_specs=[
            pl.BlockSpec((S, D), lambda t: (0, 0)),
            pl.BlockSpec((D, tile), lambda t: (0, t)),
        ],
        out_specs=pl.BlockSpec((S, tile), lambda t: (0, t)),
        out_shape=jax.ShapeDtypeStruct((S, H * PK), _F32),
    )(x2, Wcat)


def _hash_body(qkv_ref, rotT_ref, bkt_ref):
    qk = qkv_ref[:, 0:DH]
    qkT = qk.T
    rrT = jnp.dot(rotT_ref[0], qkT, preferred_element_type=_F32)
    iota = lax.broadcasted_iota(_I32, (N_BUCKETS, S), 0)
    for hh in range(N_HASHES):
        r = rrT[hh * 16:(hh + 1) * 16, :]
        full = jnp.concatenate([r, -r], axis=0)
        m = jnp.max(full, axis=0, keepdims=True)
        idx = jnp.min(jnp.where(full == m, iota, N_BUCKETS), axis=0,
                      keepdims=True)
        bkt_ref[0, hh:hh + 1, :] = idx


def _hash(qkv_sh, rotT):
    return pl.pallas_call(
        _hash_body,
        grid=(H,),
        in_specs=[
            pl.BlockSpec((S, PK), lambda h: (0, h)),
            pl.BlockSpec((1, N_BUCKETS, DH), lambda h: (h, 0, 0)),
        ],
        out_specs=pl.BlockSpec((1, N_HASHES, S), lambda h: (h, 0, 0)),
        out_shape=jax.ShapeDtypeStruct((H, N_HASHES, S), _I32),
    )(qkv_sh, rotT)


def _sort_gather_body(bkt_hbm, qkv_hbm,
                      sqkv_hbm, st_hbm, pos_hbm,
                      b_v, st_v, pos_v, idx_v, buf, sem):
    c = lax.axis_index("c")
    s = lax.axis_index("s")
    w = c * NS + s
    h = w // N_HASHES
    hh = w % N_HASHES
    pltpu.sync_copy(bkt_hbm.at[h, hh], b_v)
    iota16 = lax.broadcasted_iota(_I32, (L,), 0)

    def sort_body(i, ptr):
        bucket = i >> 7
        chunk = i & 127
        bv = b_v[pl.ds(chunk * L, L)]
        msk = bv == bucket
        cnts = jnp.where(msk, 1, 0)
        incl = plsc.cumsum(cnts)
        tgt = jnp.where(msk, ptr + incl - 1, S + iota16)
        plsc.store_scatter(st_v, [tgt], chunk * L + iota16)
        return ptr + jnp.sum(cnts)

    lax.fori_loop(0, N_BUCKETS * (S // L), sort_body, jnp.int32(0))

    def pos_body(ci, carry):
        svec = st_v[pl.ds(ci * L, L)]
        plsc.store_scatter(pos_v, [svec], ci * L + iota16)
        idx_v[pl.ds(ci * L, L)] = svec + h * S
        return carry

    lax.fori_loop(0, S // L, pos_body, 0)

    pltpu.sync_copy(st_v.at[pl.ds(0, S)], st_hbm.at[h, hh])
    pltpu.sync_copy(pos_v, pos_hbm.at[h, hh])

    def gath_body(g, carry):
        idxs = idx_v.at[pl.ds(g * 128, 128)]
        pltpu.async_copy(qkv_hbm.at[idxs], buf, sem).wait()
        pltpu.sync_copy(buf, sqkv_hbm.at[h, pl.ds(hh * S + g * 128, 128)])
        return carry

    lax.fori_loop(0, S // 128, gath_body, 0)


def _sort_gather(bkt, qkvflat):
    mesh = plsc.VectorSubcoreMesh(core_axis_name="c", subcore_axis_name="s",
                                  num_cores=NC, num_subcores=NS)
    fn = pl.kernel(
        _sort_gather_body,
        out_type=[
            jax.ShapeDtypeStruct((H, N_HASHES * S, PK), _F32),
            jax.ShapeDtypeStruct((H, N_HASHES, S), _I32),
            jax.ShapeDtypeStruct((H, N_HASHES, S), _I32),
        ],
        mesh=mesh,
        scratch_types=[
            pltpu.VMEM((S,), _I32),
            pltpu.VMEM((S + L,), _I32),
            pltpu.VMEM((S,), _I32),
            pltpu.VMEM((S,), _I32),
            pltpu.VMEM((128, PK), _F32),
            pltpu.SemaphoreType.DMA,
        ],
        compiler_params=_SC_PARAMS,
    )
    return fn(bkt, qkvflat)


_CPB = 4


def _attn_body(*refs):
    qrefs = refs[0:_CPB + 1]
    rrefs = refs[_CPB + 1:2 * _CPB + 2]
    crefs = refs[2 * _CPB + 2:3 * _CPB + 2]
    sol_ref = refs[-1]
    inv_sqrt_dh = 1.0 / (DH ** 0.5)

    def norm(t):
        return t * lax.rsqrt(jnp.mean(t * t, axis=1, keepdims=True) + 1e-6) \
            * inv_sqrt_dh

    nrm = [norm(r[0][:, 0:DH]) for r in qrefs]
    for j in range(_CPB):
        q = qrefs[j + 1][0][:, 0:DH]
        kv = jnp.concatenate([nrm[j], nrm[j + 1]], axis=0)
        bv = jnp.concatenate([qrefs[j][0][:, DH:PK],
                              qrefs[j + 1][0][:, DH:PK]], axis=0)
        dots = lax.dot_general(q, kv, (((1,), (1,)), ((), ())),
                               preferred_element_type=_F32)
        qi = crefs[j][0, 0]
        ki = jnp.concatenate([rrefs[j][0, 0], rrefs[j + 1][0, 0]],
                             axis=1)
        dots = (dots - 1e9 * (qi < ki).astype(_F32)
                - 1e5 * (qi == ki).astype(_F32))
        m = jnp.max(dots, axis=1, keepdims=True)
        expd = jnp.exp(dots - m)
        ssum = jnp.sum(expd, axis=1, keepdims=True)
        so = jnp.dot(expd, bv, preferred_element_type=_F32) / ssum
        lse = m + jnp.log(ssum)
        sol_ref[0, j, :, 0:DH] = so
        sol_ref[0, j, :, DH:PK] = jnp.broadcast_to(lse, (CHUNK_LEN, DH))


def _attention(sqkv, st_row, st_col):
    def at3(off):
        return lambda h, g: (h, (_CPB * g + off) % NCH, 0)

    def at4(off):
        return lambda h, g: (h, (_CPB * g + off) % NCH, 0, 0)

    qspecs = [pl.BlockSpec((1, CHUNK_LEN, PK), at3(off))
              for off in range(-1, _CPB)]
    rspecs = [pl.BlockSpec((1, 1, 1, CHUNK_LEN), at4(off))
              for off in range(-1, _CPB)]
    cspecs = [pl.BlockSpec((1, 1, CHUNK_LEN, 1), at4(off))
              for off in range(_CPB)]
    return pl.pallas_call(
        _attn_body,
        grid=(H, NCH // _CPB),
        in_specs=qspecs + rspecs + cspecs,
        out_specs=[
            pl.BlockSpec((1, _CPB, CHUNK_LEN, PK), lambda h, g: (h, g, 0, 0)),
        ],
        out_shape=[
            jax.ShapeDtypeStruct((H, NCH, CHUNK_LEN, PK), _F32),
        ],
    )(*([sqkv] * (_CPB + 1) + [st_row] * (_CPB + 1) + [st_col] * _CPB))[0]


def _unsort_body(sol_hbm, pos_hbm, o_hbm,
                 pos_v, idx_v, buf, sem):
    c = lax.axis_index("c")
    s = lax.axis_index("s")
    w = c * NS + s
    h = w // N_HASHES
    hh = w % N_HASHES
    pltpu.sync_copy(pos_hbm.at[h, hh], pos_v)
    base = h * (N_HASHES * S) + hh * S

    def idx_body(ci, carry):
        idx_v[pl.ds(ci * L, L)] = pos_v[pl.ds(ci * L, L)] + base
        return carry

    lax.fori_loop(0, S // L, idx_body, 0)

    def gath_body(g, carry):
        idxs = idx_v.at[pl.ds(g * 128, 128)]
        pltpu.async_copy(sol_hbm.at[idxs], buf, sem).wait()
        pltpu.sync_copy(buf, o_hbm.at[h, hh, pl.ds(g * 128, 128)])
        return carry

    lax.fori_loop(0, S // 128, gath_body, 0)


def _unsort(solflat, pos):
    mesh = plsc.VectorSubcoreMesh(core_axis_name="c", subcore_axis_name="s",
                                  num_cores=NC, num_subcores=NS)
    fn = pl.kernel(
        _unsort_body,
        out_type=[
            jax.ShapeDtypeStruct((H, N_HASHES, S, PK), _F32),
        ],
        mesh=mesh,
        scratch_types=[
            pltpu.VMEM((S,), _I32),
            pltpu.VMEM((S,), _I32),
            pltpu.VMEM((128, PK), _F32),
            pltpu.SemaphoreType.DMA,
        ],
        compiler_params=_SC_PARAMS,
    )
    return fn(solflat, pos)[0]


def _combine_body(o_ref, wout_ref, out_ref):
    h = pl.program_id(0)
    l0 = o_ref[0, 0, :, DH:DH + 1]
    l1 = o_ref[0, 1, :, DH:DH + 1]
    m = jnp.maximum(l0, l1)
    lse = m + jnp.log(jnp.exp(l0 - m) + jnp.exp(l1 - m))
    p0 = jnp.exp(l0 - lse)
    p1 = jnp.exp(l1 - lse)
    wsum = o_ref[0, 0, :, 0:DH] * p0 + o_ref[0, 1, :, 0:DH] * p1
    contrib = jnp.dot(wsum, wout_ref[0], preferred_element_type=_F32)

    @pl.when(h == 0)
    def _():
        out_ref[...] = jnp.zeros_like(out_ref)

    out_ref[...] += contrib


def _combine(o, Wout):
    return pl.pallas_call(
        _combine_body,
        grid=(H,),
        in_specs=[
            pl.BlockSpec((1, N_HASHES, S, PK), lambda h: (h, 0, 0, 0)),
            pl.BlockSpec((1, DH, D), lambda h: (h, 0, 0)),
        ],
        out_specs=pl.BlockSpec((S, D), lambda h: (0, 0)),
        out_shape=jax.ShapeDtypeStruct((S, D), _F32),
    )(o, Wout)


def kernel(x, Wqk, Wv, Wout):
    rot = jax.random.normal(jax.random.key(1),
                            (H, DH, N_HASHES, N_BUCKETS // 2),
                            dtype=_F32).reshape(H, DH, N_HASHES * 16)
    rotT = jnp.transpose(rot, (0, 2, 1))
    x2 = x[0]
    Wcat = jnp.concatenate([Wqk, Wv], axis=-1)
    Wcat = jnp.transpose(Wcat, (1, 0, 2)).reshape(D, H * PK)
    qkv_sh = _proj(x2, Wcat)
    bkt_t = _hash(qkv_sh, rotT)
    qkv_flat = jnp.transpose(qkv_sh.reshape(S, H, PK),
                             (1, 0, 2)).reshape(H * S, PK)
    sqkv, st, pos = _sort_gather(bkt_t, qkv_flat)
    stf = st.astype(_F32)
    st_row = stf.reshape(H, NCH, 1, CHUNK_LEN)
    st_col = stf.reshape(H, NCH, CHUNK_LEN, 1)
    sol = _attention(sqkv, st_row, st_col)
    o = _unsort(sol.reshape(H * N_HASHES * S, PK), pos)
    out = _combine(o, Wout)
    return out.reshape(B, S, D)


# speedup vs baseline: 7.2058x; 1.1888x over previous
import jax
import jax.numpy as jnp
from jax import lax
from jax.experimental import pallas as pl
from jax.experimental.pallas import tpu as pltpu
from jax.experimental.pallas import tpu_sc as plsc

B, S, D = 1, 2048, 1024
H, DH = 16, 64
CHUNK_LEN = 128
N_HASHES = 2
N_BUCKETS = 32
NCH = (N_HASHES * S) // CHUNK_LEN
NC, NS, L = 2, 16, 16
PK = 2 * DH

_F32 = jnp.float32
_I32 = jnp.int32
_SC_PARAMS = pltpu.CompilerParams(needs_layout_passes=False)


def _proj_body(x_ref, w_ref, out_ref):
    out_ref[...] = jnp.dot(x_ref[...], w_ref[...],
                           preferred_element_type=_F32)


def _proj(x2, Wcat):
    ntile = 4
    tile = H * PK // ntile
    return pl.pallas_call(
        _proj_body,
        grid=(ntile,),
        in_specs=[
            pl.BlockSpec((S, D), lambda t: (0, 0)),
            pl.BlockSpec((D, tile), lambda t: (0, t)),
        ],
        out_specs=pl.BlockSpec((S, tile), lambda t: (0, t)),
        out_shape=jax.ShapeDtypeStruct((S, H * PK), _F32),
    )(x2, Wcat)


def _hash_body(qkv_ref, rotT_ref, bkt_ref):
    qk = qkv_ref[:, 0:DH]
    qkT = qk.T
    rrT = jnp.dot(rotT_ref[0], qkT, preferred_element_type=_F32)
    iota = lax.broadcasted_iota(_I32, (N_BUCKETS, S), 0)
    for hh in range(N_HASHES):
        r = rrT[hh * 16:(hh + 1) * 16, :]
        full = jnp.concatenate([r, -r], axis=0)
        m = jnp.max(full, axis=0, keepdims=True)
        idx = jnp.min(jnp.where(full == m, iota, N_BUCKETS), axis=0,
                      keepdims=True)
        bkt_ref[0, hh:hh + 1, :] = idx


def _hash(qkv_sh, rotT):
    return pl.pallas_call(
        _hash_body,
        grid=(H,),
        in_specs=[
            pl.BlockSpec((S, PK), lambda h: (0, h)),
            pl.BlockSpec((1, N_BUCKETS, DH), lambda h: (h, 0, 0)),
        ],
        out_specs=pl.BlockSpec((1, N_HASHES, S), lambda h: (h, 0, 0)),
        out_shape=jax.ShapeDtypeStruct((H, N_HASHES, S), _I32),
    )(qkv_sh, rotT)


_NG = 8
_GR = S // _NG


def _sort_gather_body(bkt_hbm, qkv_hbm,
                      sqkv_hbm, st_hbm, pos_hbm,
                      b_v, st_v, pos_v, idx_v, rank_v, cnt_v,
                      buf0, buf1, gsem0, gsem1):
    c = lax.axis_index("c")
    s = lax.axis_index("s")
    w = c * NS + s
    h = w // N_HASHES
    hh = w % N_HASHES
    pltpu.sync_copy(bkt_hbm.at[h, hh], b_v)
    iota16 = lax.broadcasted_iota(_I32, (L,), 0)
    zero16 = jnp.zeros((L,), _I32)
    cnt_v[pl.ds(0, L)] = zero16
    cnt_v[pl.ds(L, L)] = zero16

    def rank_body(ci, carry):
        bv = b_v[pl.ds(ci * L, L)]
        base = plsc.load_gather(cnt_v, [bv])
        run, last = plsc.scan_count(bv)
        rank_v[pl.ds(ci * L, L)] = base + run - 1
        plsc.store_scatter(cnt_v, [bv], base + run, mask=last)
        return carry

    lax.fori_loop(0, S // L, rank_body, 0)

    c0 = cnt_v[pl.ds(0, L)]
    c1 = cnt_v[pl.ds(L, L)]
    e0 = plsc.cumsum(c0) - c0
    e1 = plsc.cumsum(c1) - c1 + jnp.sum(c0)
    cnt_v[pl.ds(0, L)] = e0
    cnt_v[pl.ds(L, L)] = e1

    def pos_body(ci, carry):
        bv = b_v[pl.ds(ci * L, L)]
        off = plsc.load_gather(cnt_v, [bv])
        p = off + rank_v[pl.ds(ci * L, L)]
        pos_v[pl.ds(ci * L, L)] = p
        plsc.store_scatter(st_v, [p], ci * L + iota16)
        return carry

    lax.fori_loop(0, S // L, pos_body, 0)

    def idx_body(ci, carry):
        idx_v[pl.ds(ci * L, L)] = st_v[pl.ds(ci * L, L)] + h * S
        return carry

    lax.fori_loop(0, S // L, idx_body, 0)

    pltpu.sync_copy(st_v, st_hbm.at[h, hh])
    pltpu.sync_copy(pos_v, pos_hbm.at[h, hh])

    bufs = (buf0, buf1)
    sems = (gsem0, gsem1)
    gh = [None] * _NG
    for k in range(_NG):
        gh[k] = pltpu.async_copy(
            qkv_hbm.at[idx_v.at[pl.ds(k * _GR, _GR)]], bufs[k % 2],
            sems[k % 2])
        if k >= 1:
            gh[k - 1].wait()
            pltpu.sync_copy(bufs[(k - 1) % 2],
                            sqkv_hbm.at[h, pl.ds(hh * S + (k - 1) * _GR, _GR)])
    gh[_NG - 1].wait()
    pltpu.sync_copy(bufs[(_NG - 1) % 2],
                    sqkv_hbm.at[h, pl.ds(hh * S + (_NG - 1) * _GR, _GR)])


def _sort_gather(bkt, qkvflat):
    mesh = plsc.VectorSubcoreMesh(core_axis_name="c", subcore_axis_name="s",
                                  num_cores=NC, num_subcores=NS)
    fn = pl.kernel(
        _sort_gather_body,
        out_type=[
            jax.ShapeDtypeStruct((H, N_HASHES * S, PK), _F32),
            jax.ShapeDtypeStruct((H, N_HASHES, S), _I32),
            jax.ShapeDtypeStruct((H, N_HASHES, S), _I32),
        ],
        mesh=mesh,
        scratch_types=[
            pltpu.VMEM((S,), _I32),
            pltpu.VMEM((S,), _I32),
            pltpu.VMEM((S,), _I32),
            pltpu.VMEM((S,), _I32),
            pltpu.VMEM((S,), _I32),
            pltpu.VMEM((N_BUCKETS,), _I32),
            pltpu.VMEM((_GR, PK), _F32),
            pltpu.VMEM((_GR, PK), _F32),
            pltpu.SemaphoreType.DMA,
            pltpu.SemaphoreType.DMA,
        ],
        compiler_params=_SC_PARAMS,
    )
    return fn(bkt, qkvflat)


_CPB = 4


def _attn_body(*refs):
    qrefs = refs[0:_CPB + 1]
    rrefs = refs[_CPB + 1:2 * _CPB + 2]
    crefs = refs[2 * _CPB + 2:3 * _CPB + 2]
    sol_ref = refs[-1]
    inv_sqrt_dh = 1.0 / (DH ** 0.5)

    def norm(t):
        return t * lax.rsqrt(jnp.mean(t * t, axis=1, keepdims=True) + 1e-6) \
            * inv_sqrt_dh

    nrm = [norm(r[0][:, 0:DH]) for r in qrefs]
    for j in range(_CPB):
        q = qrefs[j + 1][0][:, 0:DH]
        kv = jnp.concatenate([nrm[j], nrm[j + 1]], axis=0)
        bv = jnp.concatenate([qrefs[j][0][:, DH:PK],
                              qrefs[j + 1][0][:, DH:PK]], axis=0)
        dots = lax.dot_general(q, kv, (((1,), (1,)), ((), ())),
                               preferred_element_type=_F32)
        qi = crefs[j][0, 0]
        ki = jnp.concatenate([rrefs[j][0, 0], rrefs[j + 1][0, 0]],
                             axis=1)
        dots = (dots - 1e9 * (qi < ki).astype(_F32)
                - 1e5 * (qi == ki).astype(_F32))
        m = jnp.max(dots, axis=1, keepdims=True)
        expd = jnp.exp(dots - m)
        ssum = jnp.sum(expd, axis=1, keepdims=True)
        so = jnp.dot(expd, bv, preferred_element_type=_F32) / ssum
        lse = m + jnp.log(ssum)
        sol_ref[0, j, :, 0:DH] = so
        sol_ref[0, j, :, DH:PK] = jnp.broadcast_to(lse, (CHUNK_LEN, DH))


def _attention(sqkv, st_row, st_col):
    def at3(off):
        return lambda h, g: (h, (_CPB * g + off) % NCH, 0)

    def at4(off):
        return lambda h, g: (h, (_CPB * g + off) % NCH, 0, 0)

    qspecs = [pl.BlockSpec((1, CHUNK_LEN, PK), at3(off))
              for off in range(-1, _CPB)]
    rspecs = [pl.BlockSpec((1, 1, 1, CHUNK_LEN), at4(off))
              for off in range(-1, _CPB)]
    cspecs = [pl.BlockSpec((1, 1, CHUNK_LEN, 1), at4(off))
              for off in range(_CPB)]
    return pl.pallas_call(
        _attn_body,
        grid=(H, NCH // _CPB),
        in_specs=qspecs + rspecs + cspecs,
        out_specs=[
            pl.BlockSpec((1, _CPB, CHUNK_LEN, PK), lambda h, g: (h, g, 0, 0)),
        ],
        out_shape=[
            jax.ShapeDtypeStruct((H, NCH, CHUNK_LEN, PK), _F32),
        ],
    )(*([sqkv] * (_CPB + 1) + [st_row] * (_CPB + 1) + [st_col] * _CPB))[0]


def _unsort_body(sol_hbm, pos_hbm, o_hbm,
                 pos_v, idx_v, buf0, buf1, gsem0, gsem1):
    c = lax.axis_index("c")
    s = lax.axis_index("s")
    w = c * NS + s
    h = w // N_HASHES
    hh = w % N_HASHES
    pltpu.sync_copy(pos_hbm.at[h, hh], pos_v)
    base = h * (N_HASHES * S) + hh * S

    def idx_body(ci, carry):
        idx_v[pl.ds(ci * L, L)] = pos_v[pl.ds(ci * L, L)] + base
        return carry

    lax.fori_loop(0, S // L, idx_body, 0)

    bufs = (buf0, buf1)
    sems = (gsem0, gsem1)
    gh = [None] * _NG
    for k in range(_NG):
        gh[k] = pltpu.async_copy(
            sol_hbm.at[idx_v.at[pl.ds(k * _GR, _GR)]], bufs[k % 2],
            sems[k % 2])
        if k >= 1:
            gh[k - 1].wait()
            pltpu.sync_copy(bufs[(k - 1) % 2],
                            o_hbm.at[h, hh, pl.ds((k - 1) * _GR, _GR)])
    gh[_NG - 1].wait()
    pltpu.sync_copy(bufs[(_NG - 1) % 2],
                    o_hbm.at[h, hh, pl.ds((_NG - 1) * _GR, _GR)])


def _unsort(solflat, pos):
    mesh = plsc.VectorSubcoreMesh(core_axis_name="c", subcore_axis_name="s",
                                  num_cores=NC, num_subcores=NS)
    fn = pl.kernel(
        _unsort_body,
        out_type=[
            jax.ShapeDtypeStruct((H, N_HASHES, S, PK), _F32),
        ],
        mesh=mesh,
        scratch_types=[
            pltpu.VMEM((S,), _I32),
            pltpu.VMEM((S,), _I32),
            pltpu.VMEM((_GR, PK), _F32),
            pltpu.VMEM((_GR, PK), _F32),
            pltpu.SemaphoreType.DMA,
            pltpu.SemaphoreType.DMA,
        ],
        compiler_params=_SC_PARAMS,
    )
    return fn(solflat, pos)[0]


def _combine_body(o_ref, wout_ref, out_ref):
    h = pl.program_id(0)
    l0 = o_ref[0, 0, :, DH:DH + 1]
    l1 = o_ref[0, 1, :, DH:DH + 1]
    m = jnp.maximum(l0, l1)
    lse = m + jnp.log(jnp.exp(l0 - m) + jnp.exp(l1 - m))
    p0 = jnp.exp(l0 - lse)
    p1 = jnp.exp(l1 - lse)
    wsum = o_ref[0, 0, :, 0:DH] * p0 + o_ref[0, 1, :, 0:DH] * p1
    contrib = jnp.dot(wsum, wout_ref[0], preferred_element_type=_F32)

    @pl.when(h == 0)
    def _():
        out_ref[...] = jnp.zeros_like(out_ref)

    out_ref[...] += contrib


def _combine(o, Wout):
    return pl.pallas_call(
        _combine_body,
        grid=(H,),
        in_specs=[
            pl.BlockSpec((1, N_HASHES, S, PK), lambda h: (h, 0, 0, 0)),
            pl.BlockSpec((1, DH, D), lambda h: (h, 0, 0)),
        ],
        out_specs=pl.BlockSpec((S, D), lambda h: (0, 0)),
        out_shape=jax.ShapeDtypeStruct((S, D), _F32),
    )(o, Wout)


def kernel(x, Wqk, Wv, Wout):
    rot = jax.random.normal(jax.random.key(1),
                            (H, DH, N_HASHES, N_BUCKETS // 2),
                            dtype=_F32).reshape(H, DH, N_HASHES * 16)
    rotT = jnp.transpose(rot, (0, 2, 1))
    x2 = x[0]
    Wcat = jnp.concatenate([Wqk, Wv], axis=-1)
    Wcat = jnp.transpose(Wcat, (1, 0, 2)).reshape(D, H * PK)
    qkv_sh = _proj(x2, Wcat)
    bkt_t = _hash(qkv_sh, rotT)
    qkv_flat = jnp.transpose(qkv_sh.reshape(S, H, PK),
                             (1, 0, 2)).reshape(H * S, PK)
    sqkv, st, pos = _sort_gather(bkt_t, qkv_flat)
    stf = st.astype(_F32)
    st_row = stf.reshape(H, NCH, 1, CHUNK_LEN)
    st_col = stf.reshape(H, NCH, CHUNK_LEN, 1)
    sol = _attention(sqkv, st_row, st_col)
    o = _unsort(sol.reshape(H * N_HASHES * S, PK), pos)
    out = _combine(o, Wout)
    return out.reshape(B, S, D)


# speedup vs baseline: 7.4291x; 1.0310x over previous
import jax
import jax.numpy as jnp
from jax import lax
from jax.experimental import pallas as pl
from jax.experimental.pallas import tpu as pltpu
from jax.experimental.pallas import tpu_sc as plsc

B, S, D = 1, 2048, 1024
H, DH = 16, 64
CHUNK_LEN = 128
N_HASHES = 2
N_BUCKETS = 32
NCH = (N_HASHES * S) // CHUNK_LEN
NC, NS, L = 2, 16, 16
PK = 2 * DH

_F32 = jnp.float32
_I32 = jnp.int32
_SC_PARAMS = pltpu.CompilerParams(needs_layout_passes=False)


def _proj_body(x_ref, w_ref, out_ref):
    out_ref[...] = jnp.dot(x_ref[...], w_ref[...],
                           preferred_element_type=_F32)


def _proj(x2, Wcat):
    ntile = 4
    tile = H * PK // ntile
    return pl.pallas_call(
        _proj_body,
        grid=(ntile,),
        in_specs=[
            pl.BlockSpec((S, D), lambda t: (0, 0)),
            pl.BlockSpec((D, tile), lambda t: (0, t)),
        ],
        out_specs=pl.BlockSpec((S, tile), lambda t: (0, t)),
        out_shape=jax.ShapeDtypeStruct((S, H * PK), _F32),
    )(x2, Wcat)


def _hash_body(qkv_ref, rotT_ref, bkt_ref):
    qk = qkv_ref[:, 0:DH]
    qkT = qk.T
    rrT = jnp.dot(rotT_ref[0], qkT, preferred_element_type=_F32)
    iota = lax.broadcasted_iota(_I32, (N_BUCKETS, S), 0)
    for hh in range(N_HASHES):
        r = rrT[hh * 16:(hh + 1) * 16, :]
        full = jnp.concatenate([r, -r], axis=0)
        m = jnp.max(full, axis=0, keepdims=True)
        idx = jnp.min(jnp.where(full == m, iota, N_BUCKETS), axis=0,
                      keepdims=True)
        bkt_ref[0, hh:hh + 1, :] = idx


def _hash(qkv_sh, rotT):
    return pl.pallas_call(
        _hash_body,
        grid=(H,),
        in_specs=[
            pl.BlockSpec((S, PK), lambda h: (0, h)),
            pl.BlockSpec((1, N_BUCKETS, DH), lambda h: (h, 0, 0)),
        ],
        out_specs=pl.BlockSpec((1, N_HASHES, S), lambda h: (h, 0, 0)),
        out_shape=jax.ShapeDtypeStruct((H, N_HASHES, S), _I32),
    )(qkv_sh, rotT)


_NG = 8
_GR = S // _NG


def _sort_gather_body(bkt_hbm, qkv_hbm,
                      sqkv_hbm, st_hbm, pos_hbm,
                      b_v, st_v, pos_v, idx_v, rank_v, cnt_v,
                      buf0, buf1, gsem0, gsem1):
    c = lax.axis_index("c")
    s = lax.axis_index("s")
    w = c * NS + s
    h = w // N_HASHES
    hh = w % N_HASHES
    pltpu.sync_copy(bkt_hbm.at[h, hh], b_v)
    iota16 = lax.broadcasted_iota(_I32, (L,), 0)
    zero16 = jnp.zeros((L,), _I32)
    cnt_v[pl.ds(0, L)] = zero16
    cnt_v[pl.ds(L, L)] = zero16

    def rank_body(ci, carry):
        bv = b_v[pl.ds(ci * L, L)]
        base = plsc.load_gather(cnt_v, [bv])
        run, last = plsc.scan_count(bv)
        rank_v[pl.ds(ci * L, L)] = base + run - 1
        plsc.store_scatter(cnt_v, [bv], base + run, mask=last)
        return carry

    lax.fori_loop(0, S // L, rank_body, 0)

    c0 = cnt_v[pl.ds(0, L)]
    c1 = cnt_v[pl.ds(L, L)]
    e0 = plsc.cumsum(c0) - c0
    e1 = plsc.cumsum(c1) - c1 + jnp.sum(c0)
    cnt_v[pl.ds(0, L)] = e0
    cnt_v[pl.ds(L, L)] = e1

    def pos_body(ci, carry):
        bv = b_v[pl.ds(ci * L, L)]
        off = plsc.load_gather(cnt_v, [bv])
        p = off + rank_v[pl.ds(ci * L, L)]
        pos_v[pl.ds(ci * L, L)] = p
        plsc.store_scatter(st_v, [p], ci * L + iota16)
        return carry

    lax.fori_loop(0, S // L, pos_body, 0)

    def idx_body(ci, carry):
        idx_v[pl.ds(ci * L, L)] = st_v[pl.ds(ci * L, L)] + h * S
        return carry

    lax.fori_loop(0, S // L, idx_body, 0)

    pltpu.sync_copy(st_v, st_hbm.at[h, hh])
    pltpu.sync_copy(pos_v, pos_hbm.at[h, hh])

    bufs = (buf0, buf1)
    sems = (gsem0, gsem1)
    gh = [None] * _NG
    for k in range(_NG):
        gh[k] = pltpu.async_copy(
            qkv_hbm.at[idx_v.at[pl.ds(k * _GR, _GR)]], bufs[k % 2],
            sems[k % 2])
        if k >= 1:
            gh[k - 1].wait()
            pltpu.sync_copy(bufs[(k - 1) % 2],
                            sqkv_hbm.at[h, pl.ds(hh * S + (k - 1) * _GR, _GR)])
    gh[_NG - 1].wait()
    pltpu.sync_copy(bufs[(_NG - 1) % 2],
                    sqkv_hbm.at[h, pl.ds(hh * S + (_NG - 1) * _GR, _GR)])


def _sort_gather(bkt, qkvflat):
    mesh = plsc.VectorSubcoreMesh(core_axis_name="c", subcore_axis_name="s",
                                  num_cores=NC, num_subcores=NS)
    fn = pl.kernel(
        _sort_gather_body,
        out_type=[
            jax.ShapeDtypeStruct((H, N_HASHES * S, PK), _F32),
            jax.ShapeDtypeStruct((H, N_HASHES, S), _I32),
            jax.ShapeDtypeStruct((H, N_HASHES, S), _I32),
        ],
        mesh=mesh,
        scratch_types=[
            pltpu.VMEM((S,), _I32),
            pltpu.VMEM((S,), _I32),
            pltpu.VMEM((S,), _I32),
            pltpu.VMEM((S,), _I32),
            pltpu.VMEM((S,), _I32),
            pltpu.VMEM((N_BUCKETS,), _I32),
            pltpu.VMEM((_GR, PK), _F32),
            pltpu.VMEM((_GR, PK), _F32),
            pltpu.SemaphoreType.DMA,
            pltpu.SemaphoreType.DMA,
        ],
        compiler_params=_SC_PARAMS,
    )
    return fn(bkt, qkvflat)


_CPB = 8


def _attn_body(*refs):
    qrefs = refs[0:_CPB + 1]
    rrefs = refs[_CPB + 1:2 * _CPB + 2]
    crefs = refs[2 * _CPB + 2:3 * _CPB + 2]
    sol_ref = refs[-1]
    inv_sqrt_dh = 1.0 / (DH ** 0.5)

    def norm(t):
        return t * lax.rsqrt(jnp.mean(t * t, axis=1, keepdims=True) + 1e-6) \
            * inv_sqrt_dh

    nrm = [norm(r[0][:, 0:DH]) for r in qrefs]
    for j in range(_CPB):
        q = qrefs[j + 1][0][:, 0:DH]
        kv = jnp.concatenate([nrm[j], nrm[j + 1]], axis=0)
        bv = jnp.concatenate([qrefs[j][0][:, DH:PK],
                              qrefs[j + 1][0][:, DH:PK]], axis=0)
        dots = lax.dot_general(q, kv, (((1,), (1,)), ((), ())),
                               preferred_element_type=_F32)
        qi = crefs[j][0, 0]
        ki = jnp.concatenate([rrefs[j][0, 0], rrefs[j + 1][0, 0]],
                             axis=1)
        dots = (dots - 1e9 * (qi < ki).astype(_F32)
                - 1e5 * (qi == ki).astype(_F32))
        m = jnp.max(dots, axis=1, keepdims=True)
        expd = jnp.exp(dots - m)
        ssum = jnp.sum(expd, axis=1, keepdims=True)
        recip = 1.0 / ssum
        so = jnp.dot(expd, bv, preferred_element_type=_F32) * recip
        lse = m + jnp.log(ssum)
        sol_ref[0, j, :, 0:DH] = so
        sol_ref[0, j, :, DH:DH + 8] = jnp.broadcast_to(lse, (CHUNK_LEN, 8))


def _attention(sqkv, st_row, st_col):
    def at3(off):
        return lambda h, g: (h, (_CPB * g + off) % NCH, 0)

    def at4(off):
        return lambda h, g: (h, (_CPB * g + off) % NCH, 0, 0)

    qspecs = [pl.BlockSpec((1, CHUNK_LEN, PK), at3(off))
              for off in range(-1, _CPB)]
    rspecs = [pl.BlockSpec((1, 1, 1, CHUNK_LEN), at4(off))
              for off in range(-1, _CPB)]
    cspecs = [pl.BlockSpec((1, 1, CHUNK_LEN, 1), at4(off))
              for off in range(_CPB)]
    return pl.pallas_call(
        _attn_body,
        grid=(H, NCH // _CPB),
        in_specs=qspecs + rspecs + cspecs,
        out_specs=[
            pl.BlockSpec((1, _CPB, CHUNK_LEN, PK), lambda h, g: (h, g, 0, 0)),
        ],
        out_shape=[
            jax.ShapeDtypeStruct((H, NCH, CHUNK_LEN, PK), _F32),
        ],
    )(*([sqkv] * (_CPB + 1) + [st_row] * (_CPB + 1) + [st_col] * _CPB))[0]


def _unsort_body(sol_hbm, pos_hbm, o_hbm,
                 pos_v, idx_v, buf0, buf1, gsem0, gsem1):
    c = lax.axis_index("c")
    s = lax.axis_index("s")
    w = c * NS + s
    h = w // N_HASHES
    hh = w % N_HASHES
    pltpu.sync_copy(pos_hbm.at[h, hh], pos_v)
    base = h * (N_HASHES * S) + hh * S

    def idx_body(ci, carry):
        idx_v[pl.ds(ci * L, L)] = pos_v[pl.ds(ci * L, L)] + base
        return carry

    lax.fori_loop(0, S // L, idx_body, 0)

    bufs = (buf0, buf1)
    sems = (gsem0, gsem1)
    gh = [None] * _NG
    for k in range(_NG):
        gh[k] = pltpu.async_copy(
            sol_hbm.at[idx_v.at[pl.ds(k * _GR, _GR)]], bufs[k % 2],
            sems[k % 2])
        if k >= 1:
            gh[k - 1].wait()
            pltpu.sync_copy(bufs[(k - 1) % 2],
                            o_hbm.at[h, hh, pl.ds((k - 1) * _GR, _GR)])
    gh[_NG - 1].wait()
    pltpu.sync_copy(bufs[(_NG - 1) % 2],
                    o_hbm.at[h, hh, pl.ds((_NG - 1) * _GR, _GR)])


def _unsort(solflat, pos):
    mesh = plsc.VectorSubcoreMesh(core_axis_name="c", subcore_axis_name="s",
                                  num_cores=NC, num_subcores=NS)
    fn = pl.kernel(
        _unsort_body,
        out_type=[
            jax.ShapeDtypeStruct((H, N_HASHES, S, PK), _F32),
        ],
        mesh=mesh,
        scratch_types=[
            pltpu.VMEM((S,), _I32),
            pltpu.VMEM((S,), _I32),
            pltpu.VMEM((_GR, PK), _F32),
            pltpu.VMEM((_GR, PK), _F32),
            pltpu.SemaphoreType.DMA,
            pltpu.SemaphoreType.DMA,
        ],
        compiler_params=_SC_PARAMS,
    )
    return fn(solflat, pos)[0]


def _combine_body(o_ref, wout_ref, out_ref):
    h = pl.program_id(0)
    l0 = o_ref[0, 0, :, DH:DH + 1]
    l1 = o_ref[0, 1, :, DH:DH + 1]
    m = jnp.maximum(l0, l1)
    lse = m + jnp.log(jnp.exp(l0 - m) + jnp.exp(l1 - m))
    p0 = jnp.exp(l0 - lse)
    p1 = jnp.exp(l1 - lse)
    wsum = o_ref[0, 0, :, 0:DH] * p0 + o_ref[0, 1, :, 0:DH] * p1
    contrib = jnp.dot(wsum, wout_ref[0], preferred_element_type=_F32)

    @pl.when(h == 0)
    def _():
        out_ref[...] = jnp.zeros_like(out_ref)

    out_ref[...] += contrib


def _combine(o, Wout):
    return pl.pallas_call(
        _combine_body,
        grid=(H,),
        in_specs=[
            pl.BlockSpec((1, N_HASHES, S, PK), lambda h: (h, 0, 0, 0)),
            pl.BlockSpec((1, DH, D), lambda h: (h, 0, 0)),
        ],
        out_specs=pl.BlockSpec((S, D), lambda h: (0, 0)),
        out_shape=jax.ShapeDtypeStruct((S, D), _F32),
    )(o, Wout)


def kernel(x, Wqk, Wv, Wout):
    rot = jax.random.normal(jax.random.key(1),
                            (H, DH, N_HASHES, N_BUCKETS // 2),
                            dtype=_F32).reshape(H, DH, N_HASHES * 16)
    rotT = jnp.transpose(rot, (0, 2, 1))
    x2 = x[0]
    Wcat = jnp.concatenate([Wqk, Wv], axis=-1)
    Wcat = jnp.transpose(Wcat, (1, 0, 2)).reshape(D, H * PK)
    qkv_sh = _proj(x2, Wcat)
    bkt_t = _hash(qkv_sh, rotT)
    qkv_flat = jnp.transpose(qkv_sh.reshape(S, H, PK),
                             (1, 0, 2)).reshape(H * S, PK)
    sqkv, st, pos = _sort_gather(bkt_t, qkv_flat)
    stf = st.astype(_F32)
    st_row = stf.reshape(H, NCH, 1, CHUNK_LEN)
    st_col = stf.reshape(H, NCH, CHUNK_LEN, 1)
    sol = _attention(sqkv, st_row, st_col)
    o = _unsort(sol.reshape(H * N_HASHES * S, PK), pos)
    out = _combine(o, Wout)
    return out.reshape(B, S, D)


# speedup vs baseline: 7.5590x; 1.0175x over previous
import jax
import jax.numpy as jnp
from jax import lax
from jax.experimental import pallas as pl
from jax.experimental.pallas import tpu as pltpu
from jax.experimental.pallas import tpu_sc as plsc

B, S, D = 1, 2048, 1024
H, DH = 16, 64
CHUNK_LEN = 128
N_HASHES = 2
N_BUCKETS = 32
NCH = (N_HASHES * S) // CHUNK_LEN
NC, NS, L = 2, 16, 16
PK = 2 * DH

_F32 = jnp.float32
_I32 = jnp.int32
_SC_PARAMS = pltpu.CompilerParams(needs_layout_passes=False)


def _proj_body(x_ref, w_ref, out_ref):
    out_ref[...] = jnp.dot(x_ref[...], w_ref[...],
                           preferred_element_type=_F32)


def _proj(x2, Wcat):
    ntile = 4
    tile = H * PK // ntile
    return pl.pallas_call(
        _proj_body,
        grid=(ntile,),
        in_specs=[
            pl.BlockSpec((S, D), lambda t: (0, 0)),
            pl.BlockSpec((D, tile), lambda t: (0, t)),
        ],
        out_specs=pl.BlockSpec((S, tile), lambda t: (0, t)),
        out_shape=jax.ShapeDtypeStruct((S, H * PK), _F32),
    )(x2, Wcat)


def _hash_body(qkv_ref, rotT_ref, bkt_ref):
    qk = qkv_ref[:, 0:DH]
    qkT = qk.T
    rrT = jnp.dot(rotT_ref[0], qkT, preferred_element_type=_F32)
    iota = lax.broadcasted_iota(_I32, (N_BUCKETS, S), 0)
    for hh in range(N_HASHES):
        r = rrT[hh * 16:(hh + 1) * 16, :]
        full = jnp.concatenate([r, -r], axis=0)
        m = jnp.max(full, axis=0, keepdims=True)
        idx = jnp.min(jnp.where(full == m, iota, N_BUCKETS), axis=0,
                      keepdims=True)
        bkt_ref[0, hh:hh + 1, :] = idx


def _hash(qkv_sh, rotT):
    return pl.pallas_call(
        _hash_body,
        grid=(H,),
        in_specs=[
            pl.BlockSpec((S, PK), lambda h: (0, h)),
            pl.BlockSpec((1, N_BUCKETS, DH), lambda h: (h, 0, 0)),
        ],
        out_specs=pl.BlockSpec((1, N_HASHES, S), lambda h: (h, 0, 0)),
        out_shape=jax.ShapeDtypeStruct((H, N_HASHES, S), _I32),
    )(qkv_sh, rotT)


_NG = 8
_GR = S // _NG


def _sort_gather_body(bkt_hbm, qkv_hbm,
                      sqkv_hbm, st_hbm, pos_hbm,
                      b_v, st_v, pos_v, rank_v, cnt_v,
                      buf0, buf1, gsem0, gsem1):
    c = lax.axis_index("c")
    s = lax.axis_index("s")
    w = c * NS + s
    h = w // N_HASHES
    hh = w % N_HASHES
    pltpu.sync_copy(bkt_hbm.at[h, hh], b_v)
    iota16 = lax.broadcasted_iota(_I32, (L,), 0)
    zero16 = jnp.zeros((L,), _I32)
    cnt_v[pl.ds(0, L)] = zero16
    cnt_v[pl.ds(L, L)] = zero16

    def rank_body(ci, carry):
        bv = b_v[pl.ds(ci * L, L)]
        base = plsc.load_gather(cnt_v, [bv])
        run, last = plsc.scan_count(bv)
        rank_v[pl.ds(ci * L, L)] = base + run - 1
        plsc.store_scatter(cnt_v, [bv], base + run, mask=last)
        return carry

    lax.fori_loop(0, S // L, rank_body, 0)

    c0 = cnt_v[pl.ds(0, L)]
    c1 = cnt_v[pl.ds(L, L)]
    e0 = plsc.cumsum(c0) - c0
    e1 = plsc.cumsum(c1) - c1 + jnp.sum(c0)
    cnt_v[pl.ds(0, L)] = e0
    cnt_v[pl.ds(L, L)] = e1

    def pos_body(ci, carry):
        bv = b_v[pl.ds(ci * L, L)]
        off = plsc.load_gather(cnt_v, [bv])
        p = off + rank_v[pl.ds(ci * L, L)]
        pos_v[pl.ds(ci * L, L)] = p
        plsc.store_scatter(st_v, [p], ci * L + iota16)
        return carry

    lax.fori_loop(0, S // L, pos_body, 0)

    pltpu.sync_copy(st_v, st_hbm.at[h, hh])
    pltpu.sync_copy(pos_v, pos_hbm.at[h, hh])

    bufs = (buf0, buf1)
    sems = (gsem0, gsem1)
    gh = [None] * _NG
    for k in range(_NG):
        gh[k] = pltpu.async_copy(
            qkv_hbm.at[st_v.at[pl.ds(k * _GR, _GR)], pl.ds(h, 1)],
            bufs[k % 2], sems[k % 2])
        if k >= 1:
            gh[k - 1].wait()
            pltpu.sync_copy(
                bufs[(k - 1) % 2],
                sqkv_hbm.at[h, pl.ds(hh * S + (k - 1) * _GR, _GR)])
    gh[_NG - 1].wait()
    pltpu.sync_copy(bufs[(_NG - 1) % 2],
                    sqkv_hbm.at[h, pl.ds(hh * S + (_NG - 1) * _GR, _GR)])


def _sort_gather(bkt, qkvflat):
    mesh = plsc.VectorSubcoreMesh(core_axis_name="c", subcore_axis_name="s",
                                  num_cores=NC, num_subcores=NS)
    fn = pl.kernel(
        _sort_gather_body,
        out_type=[
            jax.ShapeDtypeStruct((H, N_HASHES * S, 1, PK), _F32),
            jax.ShapeDtypeStruct((H, N_HASHES, S), _I32),
            jax.ShapeDtypeStruct((H, N_HASHES, S), _I32),
        ],
        mesh=mesh,
        scratch_types=[
            pltpu.VMEM((S,), _I32),
            pltpu.VMEM((S,), _I32),
            pltpu.VMEM((S,), _I32),
            pltpu.VMEM((S,), _I32),
            pltpu.VMEM((N_BUCKETS,), _I32),
            pltpu.VMEM((_GR, 1, PK), _F32),
            pltpu.VMEM((_GR, 1, PK), _F32),
            pltpu.SemaphoreType.DMA,
            pltpu.SemaphoreType.DMA,
        ],
        compiler_params=_SC_PARAMS,
    )
    return fn(bkt, qkvflat)


_CPB = 8


def _attn_body(*refs):
    qrefs = refs[0:_CPB + 1]
    rrefs = refs[_CPB + 1:2 * _CPB + 2]
    crefs = refs[2 * _CPB + 2:3 * _CPB + 2]
    sol_ref = refs[-1]
    inv_sqrt_dh = 1.0 / (DH ** 0.5)

    def norm(t):
        return t * lax.rsqrt(jnp.mean(t * t, axis=1, keepdims=True) + 1e-6) \
            * inv_sqrt_dh

    nrm = [norm(r[0][:, 0:DH]) for r in qrefs]
    for j in range(_CPB):
        q = qrefs[j + 1][0][:, 0:DH]
        kv = jnp.concatenate([nrm[j], nrm[j + 1]], axis=0)
        bv = jnp.concatenate([qrefs[j][0][:, DH:PK],
                              qrefs[j + 1][0][:, DH:PK]], axis=0)
        dots = lax.dot_general(q, kv, (((1,), (1,)), ((), ())),
                               preferred_element_type=_F32)
        qi = crefs[j][0, 0]
        ki = jnp.concatenate([rrefs[j][0, 0], rrefs[j + 1][0, 0]],
                             axis=1)
        dots = dots + jnp.where(qi < ki, _F32(-1e9),
                                jnp.where(qi == ki, _F32(-1e5), _F32(0.0)))
        m = jnp.max(dots, axis=1, keepdims=True)
        expd = jnp.exp(dots - m)
        ssum = jnp.sum(expd, axis=1, keepdims=True)
        recip = 1.0 / ssum
        so = jnp.dot(expd, bv, preferred_element_type=_F32) * recip
        lse = m + jnp.log(ssum)
        sol_ref[0, j, :, 0:DH] = so
        sol_ref[0, j, :, DH:DH + 8] = jnp.broadcast_to(lse, (CHUNK_LEN, 8))


def _attention(sqkv, st_row, st_col):
    def at3(off):
        return lambda h, g: (h, (_CPB * g + off) % NCH, 0)

    def at4(off):
        return lambda h, g: (h, (_CPB * g + off) % NCH, 0, 0)

    qspecs = [pl.BlockSpec((1, CHUNK_LEN, PK), at3(off))
              for off in range(-1, _CPB)]
    rspecs = [pl.BlockSpec((1, 1, 1, CHUNK_LEN), at4(off))
              for off in range(-1, _CPB)]
    cspecs = [pl.BlockSpec((1, 1, CHUNK_LEN, 1), at4(off))
              for off in range(_CPB)]
    return pl.pallas_call(
        _attn_body,
        grid=(H, NCH // _CPB),
        in_specs=qspecs + rspecs + cspecs,
        out_specs=[
            pl.BlockSpec((1, _CPB, CHUNK_LEN, PK), lambda h, g: (h, g, 0, 0)),
        ],
        out_shape=[
            jax.ShapeDtypeStruct((H, NCH, CHUNK_LEN, PK), _F32),
        ],
    )(*([sqkv] * (_CPB + 1) + [st_row] * (_CPB + 1) + [st_col] * _CPB))[0]


def _unsort_body(sol_hbm, pos_hbm, o_hbm,
                 pos_v, idx_v, buf0, buf1, gsem0, gsem1):
    c = lax.axis_index("c")
    s = lax.axis_index("s")
    w = c * NS + s
    h = w // N_HASHES
    hh = w % N_HASHES
    pltpu.sync_copy(pos_hbm.at[h, hh], pos_v)
    base = h * (N_HASHES * S) + hh * S

    def idx_body(ci, carry):
        idx_v[pl.ds(ci * L, L)] = pos_v[pl.ds(ci * L, L)] + base
        return carry

    lax.fori_loop(0, S // L, idx_body, 0)

    bufs = (buf0, buf1)
    sems = (gsem0, gsem1)
    gh = [None] * _NG
    for k in range(_NG):
        gh[k] = pltpu.async_copy(
            sol_hbm.at[idx_v.at[pl.ds(k * _GR, _GR)]], bufs[k % 2],
            sems[k % 2])
        if k >= 1:
            gh[k - 1].wait()
            pltpu.sync_copy(bufs[(k - 1) % 2],
                            o_hbm.at[h, hh, pl.ds((k - 1) * _GR, _GR)])
    gh[_NG - 1].wait()
    pltpu.sync_copy(bufs[(_NG - 1) % 2],
                    o_hbm.at[h, hh, pl.ds((_NG - 1) * _GR, _GR)])


def _unsort(solflat, pos):
    mesh = plsc.VectorSubcoreMesh(core_axis_name="c", subcore_axis_name="s",
                                  num_cores=NC, num_subcores=NS)
    fn = pl.kernel(
        _unsort_body,
        out_type=[
            jax.ShapeDtypeStruct((H, N_HASHES, S, PK), _F32),
        ],
        mesh=mesh,
        scratch_types=[
            pltpu.VMEM((S,), _I32),
            pltpu.VMEM((S,), _I32),
            pltpu.VMEM((_GR, PK), _F32),
            pltpu.VMEM((_GR, PK), _F32),
            pltpu.SemaphoreType.DMA,
            pltpu.SemaphoreType.DMA,
        ],
        compiler_params=_SC_PARAMS,
    )
    return fn(solflat, pos)[0]


def _combine_body(o_ref, wout_ref, out_ref):
    h = pl.program_id(0)
    l0 = o_ref[0, 0, :, DH:DH + 1]
    l1 = o_ref[0, 1, :, DH:DH + 1]
    m = jnp.maximum(l0, l1)
    lse = m + jnp.log(jnp.exp(l0 - m) + jnp.exp(l1 - m))
    p0 = jnp.exp(l0 - lse)
    p1 = jnp.exp(l1 - lse)
    wsum = o_ref[0, 0, :, 0:DH] * p0 + o_ref[0, 1, :, 0:DH] * p1
    contrib = jnp.dot(wsum, wout_ref[0], preferred_element_type=_F32)

    @pl.when(h == 0)
    def _():
        out_ref[...] = jnp.zeros_like(out_ref)

    out_ref[...] += contrib


def _combine(o, Wout):
    return pl.pallas_call(
        _combine_body,
        grid=(H,),
        in_specs=[
            pl.BlockSpec((1, N_HASHES, S, PK), lambda h: (h, 0, 0, 0)),
            pl.BlockSpec((1, DH, D), lambda h: (h, 0, 0)),
        ],
        out_specs=pl.BlockSpec((S, D), lambda h: (0, 0)),
        out_shape=jax.ShapeDtypeStruct((S, D), _F32),
    )(o, Wout)


def kernel(x, Wqk, Wv, Wout):
    rot = jax.random.normal(jax.random.key(1),
                            (H, DH, N_HASHES, N_BUCKETS // 2),
                            dtype=_F32).reshape(H, DH, N_HASHES * 16)
    rotT = jnp.transpose(rot, (0, 2, 1))
    x2 = x[0]
    Wcat = jnp.concatenate([Wqk, Wv], axis=-1)
    Wcat = jnp.transpose(Wcat, (1, 0, 2)).reshape(D, H * PK)
    qkv_sh = _proj(x2, Wcat)
    bkt_t = _hash(qkv_sh, rotT)
    sqkv, st, pos = _sort_gather(bkt_t, qkv_sh.reshape(S, H, PK))
    sqkv = sqkv.reshape(H, N_HASHES * S, PK)
    stf = st.astype(_F32)
    st_row = stf.reshape(H, NCH, 1, CHUNK_LEN)
    st_col = stf.reshape(H, NCH, CHUNK_LEN, 1)
    sol = _attention(sqkv, st_row, st_col)
    o = _unsort(sol.reshape(H * N_HASHES * S, PK), pos)
    out = _combine(o, Wout)
    return out.reshape(B, S, D)


# speedup vs baseline: 7.5839x; 1.0033x over previous
import jax
import jax.numpy as jnp
from jax import lax
from jax.experimental import pallas as pl
from jax.experimental.pallas import tpu as pltpu
from jax.experimental.pallas import tpu_sc as plsc

B, S, D = 1, 2048, 1024
H, DH = 16, 64
CHUNK_LEN = 128
N_HASHES = 2
N_BUCKETS = 32
NCH = (N_HASHES * S) // CHUNK_LEN
NC, NS, L = 2, 16, 16
PK = 2 * DH

_F32 = jnp.float32
_I32 = jnp.int32
_SC_PARAMS = pltpu.CompilerParams(needs_layout_passes=False)


_HPT = 4


def _proj_body(x_ref, w_ref, rotT_ref, out_ref, bkt_ref):
    qkv = jnp.dot(x_ref[...], w_ref[...], preferred_element_type=_F32)
    out_ref[...] = qkv
    iota = lax.broadcasted_iota(_I32, (N_BUCKETS, S), 0)
    for j in range(_HPT):
        qkT = qkv[:, j * PK:j * PK + DH].T
        rrT = jnp.dot(rotT_ref[j], qkT, preferred_element_type=_F32)
        for hh in range(N_HASHES):
            r = rrT[hh * 16:(hh + 1) * 16, :]
            full = jnp.concatenate([r, -r], axis=0)
            m = jnp.max(full, axis=0, keepdims=True)
            idx = jnp.min(jnp.where(full == m, iota, N_BUCKETS), axis=0,
                          keepdims=True)
            bkt_ref[j, hh:hh + 1, :] = idx


def _proj_hash(x2, Wcat, rotT):
    ntile = H // _HPT
    tile = _HPT * PK
    return pl.pallas_call(
        _proj_body,
        grid=(ntile,),
        in_specs=[
            pl.BlockSpec((S, D), lambda t: (0, 0)),
            pl.BlockSpec((D, tile), lambda t: (0, t)),
            pl.BlockSpec((_HPT, N_BUCKETS, DH), lambda t: (t, 0, 0)),
        ],
        out_specs=[
            pl.BlockSpec((S, tile), lambda t: (0, t)),
            pl.BlockSpec((_HPT, N_HASHES, S), lambda t: (t, 0, 0)),
        ],
        out_shape=[
            jax.ShapeDtypeStruct((S, H * PK), _F32),
            jax.ShapeDtypeStruct((H, N_HASHES, S), _I32),
        ],
    )(x2, Wcat, rotT)


_NG = 8
_GR = S // _NG


def _sort_gather_body(bkt_hbm, qkv_hbm,
                      sqkv_hbm, st_hbm, pos_hbm,
                      b_v, st_v, pos_v, rank_v, cnt_v,
                      buf0, buf1, gsem0, gsem1):
    c = lax.axis_index("c")
    s = lax.axis_index("s")
    w = c * NS + s
    h = w // N_HASHES
    hh = w % N_HASHES
    pltpu.sync_copy(bkt_hbm.at[h, hh], b_v)
    iota16 = lax.broadcasted_iota(_I32, (L,), 0)
    zero16 = jnp.zeros((L,), _I32)
    cnt_v[pl.ds(0, L)] = zero16
    cnt_v[pl.ds(L, L)] = zero16

    def rank_body(ci, carry):
        bv = b_v[pl.ds(ci * L, L)]
        base = plsc.load_gather(cnt_v, [bv])
        run, last = plsc.scan_count(bv)
        rank_v[pl.ds(ci * L, L)] = base + run - 1
        plsc.store_scatter(cnt_v, [bv], base + run, mask=last)
        return carry

    lax.fori_loop(0, S // L, rank_body, 0)

    c0 = cnt_v[pl.ds(0, L)]
    c1 = cnt_v[pl.ds(L, L)]
    e0 = plsc.cumsum(c0) - c0
    e1 = plsc.cumsum(c1) - c1 + jnp.sum(c0)
    cnt_v[pl.ds(0, L)] = e0
    cnt_v[pl.ds(L, L)] = e1

    def pos_body(ci, carry):
        bv = b_v[pl.ds(ci * L, L)]
        off = plsc.load_gather(cnt_v, [bv])
        p = off + rank_v[pl.ds(ci * L, L)]
        pos_v[pl.ds(ci * L, L)] = p
        plsc.store_scatter(st_v, [p], ci * L + iota16)
        return carry

    lax.fori_loop(0, S // L, pos_body, 0)

    pltpu.sync_copy(st_v, st_hbm.at[h, hh])
    pltpu.sync_copy(pos_v, pos_hbm.at[h, hh])

    bufs = (buf0, buf1)
    sems = (gsem0, gsem1)
    gh = [None] * _NG
    for k in range(_NG):
        gh[k] = pltpu.async_copy(
            qkv_hbm.at[st_v.at[pl.ds(k * _GR, _GR)], pl.ds(h, 1)],
            bufs[k % 2], sems[k % 2])
        if k >= 1:
            gh[k - 1].wait()
            pltpu.sync_copy(
                bufs[(k - 1) % 2],
                sqkv_hbm.at[h, pl.ds(hh * S + (k - 1) * _GR, _GR)])
    gh[_NG - 1].wait()
    pltpu.sync_copy(bufs[(_NG - 1) % 2],
                    sqkv_hbm.at[h, pl.ds(hh * S + (_NG - 1) * _GR, _GR)])


def _sort_gather(bkt, qkvflat):
    mesh = plsc.VectorSubcoreMesh(core_axis_name="c", subcore_axis_name="s",
                                  num_cores=NC, num_subcores=NS)
    fn = pl.kernel(
        _sort_gather_body,
        out_type=[
            jax.ShapeDtypeStruct((H, N_HASHES * S, 1, PK), _F32),
            jax.ShapeDtypeStruct((H, N_HASHES, S), _I32),
            jax.ShapeDtypeStruct((H, N_HASHES, S), _I32),
        ],
        mesh=mesh,
        scratch_types=[
            pltpu.VMEM((S,), _I32),
            pltpu.VMEM((S,), _I32),
            pltpu.VMEM((S,), _I32),
            pltpu.VMEM((S,), _I32),
            pltpu.VMEM((N_BUCKETS,), _I32),
            pltpu.VMEM((_GR, 1, PK), _F32),
            pltpu.VMEM((_GR, 1, PK), _F32),
            pltpu.SemaphoreType.DMA,
            pltpu.SemaphoreType.DMA,
        ],
        compiler_params=_SC_PARAMS,
    )
    return fn(bkt, qkvflat)


_CPB = 8


def _attn_body(*refs):
    qrefs = refs[0:_CPB + 1]
    rrefs = refs[_CPB + 1:2 * _CPB + 2]
    crefs = refs[2 * _CPB + 2:3 * _CPB + 2]
    sol_ref = refs[-1]
    inv_sqrt_dh = 1.0 / (DH ** 0.5)

    def norm(t):
        return t * lax.rsqrt(jnp.mean(t * t, axis=1, keepdims=True) + 1e-6) \
            * inv_sqrt_dh

    nrm = [norm(r[0][:, 0:DH]) for r in qrefs]
    for j in range(_CPB):
        q = qrefs[j + 1][0][:, 0:DH]
        kv = jnp.concatenate([nrm[j], nrm[j + 1]], axis=0)
        bv = jnp.concatenate([qrefs[j][0][:, DH:PK],
                              qrefs[j + 1][0][:, DH:PK]], axis=0)
        dots = lax.dot_general(q, kv, (((1,), (1,)), ((), ())),
                               preferred_element_type=_F32)
        qi = crefs[j][0, 0]
        ki = jnp.concatenate([rrefs[j][0, 0], rrefs[j + 1][0, 0]],
                             axis=1)
        dots = dots + jnp.where(qi < ki, _F32(-1e9),
                                jnp.where(qi == ki, _F32(-1e5), _F32(0.0)))
        m = jnp.max(dots, axis=1, keepdims=True)
        expd = jnp.exp(dots - m)
        ssum = jnp.sum(expd, axis=1, keepdims=True)
        recip = 1.0 / ssum
        so = jnp.dot(expd, bv, preferred_element_type=_F32) * recip
        lse = m + jnp.log(ssum)
        sol_ref[0, j, :, 0:DH] = so
        sol_ref[0, j, :, DH:DH + 8] = jnp.broadcast_to(lse, (CHUNK_LEN, 8))


def _attention(sqkv, st_row, st_col):
    def at3(off):
        return lambda h, g: (h, (_CPB * g + off) % NCH, 0)

    def at4(off):
        return lambda h, g: (h, (_CPB * g + off) % NCH, 0, 0)

    qspecs = [pl.BlockSpec((1, CHUNK_LEN, PK), at3(off))
              for off in range(-1, _CPB)]
    rspecs = [pl.BlockSpec((1, 1, 1, CHUNK_LEN), at4(off))
              for off in range(-1, _CPB)]
    cspecs = [pl.BlockSpec((1, 1, CHUNK_LEN, 1), at4(off))
              for off in range(_CPB)]
    return pl.pallas_call(
        _attn_body,
        grid=(H, NCH // _CPB),
        in_specs=qspecs + rspecs + cspecs,
        out_specs=[
            pl.BlockSpec((1, _CPB, CHUNK_LEN, PK), lambda h, g: (h, g, 0, 0)),
        ],
        out_shape=[
            jax.ShapeDtypeStruct((H, NCH, CHUNK_LEN, PK), _F32),
        ],
    )(*([sqkv] * (_CPB + 1) + [st_row] * (_CPB + 1) + [st_col] * _CPB))[0]


def _unsort_body(sol_hbm, pos_hbm, o_hbm,
                 pos_v, idx_v, buf0, buf1, gsem0, gsem1):
    c = lax.axis_index("c")
    s = lax.axis_index("s")
    w = c * NS + s
    h = w // N_HASHES
    hh = w % N_HASHES
    pltpu.sync_copy(pos_hbm.at[h, hh], pos_v)
    base = h * (N_HASHES * S) + hh * S

    def idx_body(ci, carry):
        idx_v[pl.ds(ci * L, L)] = pos_v[pl.ds(ci * L, L)] + base
        return carry

    lax.fori_loop(0, S // L, idx_body, 0)

    bufs = (buf0, buf1)
    sems = (gsem0, gsem1)
    gh = [None] * _NG
    for k in range(_NG):
        gh[k] = pltpu.async_copy(
            sol_hbm.at[idx_v.at[pl.ds(k * _GR, _GR)]], bufs[k % 2],
            sems[k % 2])
        if k >= 1:
            gh[k - 1].wait()
            pltpu.sync_copy(bufs[(k - 1) % 2],
                            o_hbm.at[h, hh, pl.ds((k - 1) * _GR, _GR)])
    gh[_NG - 1].wait()
    pltpu.sync_copy(bufs[(_NG - 1) % 2],
                    o_hbm.at[h, hh, pl.ds((_NG - 1) * _GR, _GR)])


def _unsort(solflat, pos):
    mesh = plsc.VectorSubcoreMesh(core_axis_name="c", subcore_axis_name="s",
                                  num_cores=NC, num_subcores=NS)
    fn = pl.kernel(
        _unsort_body,
        out_type=[
            jax.ShapeDtypeStruct((H, N_HASHES, S, PK), _F32),
        ],
        mesh=mesh,
        scratch_types=[
            pltpu.VMEM((S,), _I32),
            pltpu.VMEM((S,), _I32),
            pltpu.VMEM((_GR, PK), _F32),
            pltpu.VMEM((_GR, PK), _F32),
            pltpu.SemaphoreType.DMA,
            pltpu.SemaphoreType.DMA,
        ],
        compiler_params=_SC_PARAMS,
    )
    return fn(solflat, pos)[0]


def _combine_body(o_ref, wout_ref, out_ref):
    h = pl.program_id(0)
    l0 = o_ref[0, 0, :, DH:DH + 1]
    l1 = o_ref[0, 1, :, DH:DH + 1]
    m = jnp.maximum(l0, l1)
    lse = m + jnp.log(jnp.exp(l0 - m) + jnp.exp(l1 - m))
    p0 = jnp.exp(l0 - lse)
    p1 = jnp.exp(l1 - lse)
    wsum = o_ref[0, 0, :, 0:DH] * p0 + o_ref[0, 1, :, 0:DH] * p1
    contrib = jnp.dot(wsum, wout_ref[0], preferred_element_type=_F32)

    @pl.when(h == 0)
    def _():
        out_ref[...] = jnp.zeros_like(out_ref)

    out_ref[...] += contrib


def _combine(o, Wout):
    return pl.pallas_call(
        _combine_body,
        grid=(H,),
        in_specs=[
            pl.BlockSpec((1, N_HASHES, S, PK), lambda h: (h, 0, 0, 0)),
            pl.BlockSpec((1, DH, D), lambda h: (h, 0, 0)),
        ],
        out_specs=pl.BlockSpec((S, D), lambda h: (0, 0)),
        out_shape=jax.ShapeDtypeStruct((S, D), _F32),
    )(o, Wout)


def kernel(x, Wqk, Wv, Wout):
    rot = jax.random.normal(jax.random.key(1),
                            (H, DH, N_HASHES, N_BUCKETS // 2),
                            dtype=_F32).reshape(H, DH, N_HASHES * 16)
    rotT = jnp.transpose(rot, (0, 2, 1))
    x2 = x[0]
    Wcat = jnp.concatenate([Wqk, Wv], axis=-1)
    Wcat = jnp.transpose(Wcat, (1, 0, 2)).reshape(D, H * PK)
    qkv_sh, bkt_t = _proj_hash(x2, Wcat, rotT)
    sqkv, st, pos = _sort_gather(bkt_t, qkv_sh.reshape(S, H, PK))
    sqkv = sqkv.reshape(H, N_HASHES * S, PK)
    stf = st.astype(_F32)
    st_row = stf.reshape(H, NCH, 1, CHUNK_LEN)
    st_col = stf.reshape(H, NCH, CHUNK_LEN, 1)
    sol = _attention(sqkv, st_row, st_col)
    o = _unsort(sol.reshape(H * N_HASHES * S, PK), pos)
    out = _combine(o, Wout)
    return out.reshape(B, S, D)
